# baseline (speedup 1.0000x reference)
import functools

import jax
from jax import lax
import jax.numpy as jnp
from jax.experimental import pallas as pl
from jax.experimental.pallas import tpu as pltpu
from jax.experimental.pallas import tpu_sc as plsc

B = 1024
NIN = 2048
NHIDDEN = 8192
NOUT = 2048
K = 64

HBE = 1024
BRM = 256
KBD = 1024


def _enc_body(x_ref, w_ref, b_ref, o_ref):
    acc = jax.lax.dot_general(
        x_ref[...], w_ref[...], (((1,), (1,)), ((), ())),
        preferred_element_type=jnp.float32)
    o_ref[...] = acc + b_ref[...]


def _key(h):
    iv = jax.lax.bitcast_convert_type(h, jnp.uint32)
    return jnp.where((iv >> 31) != 0, ~iv, iv | jnp.uint32(0x80000000))


NC = 2
NS = 16
NW = NC * NS
RPW = B // NW
NV = NHIDDEN // 16


def _splat(v):
    return jnp.full((16,), v, jnp.int32)


def _sc_key16(v):
    iv = lax.bitcast_convert_type(v, jnp.uint32)
    return jnp.where((iv >> 31) != 0, ~iv, iv | jnp.uint32(0x80000000))


def _sc_thr_body(h_hbm, thr_hbm, row_v, hist_v, cand_v, thr_v):
    wid = lax.axis_index("s") * NC + lax.axis_index("c")
    base = wid * RPW
    lane = lax.iota(jnp.int32, 16)
    ones = _splat(1)
    zeros = _splat(0)

    def do_row(r, carry_unused):
        pltpu.sync_copy(h_hbm.at[pl.ds((base + r) * NHIDDEN, NHIDDEN)], row_v)
        for g in range(16):
            hist_v[pl.ds(g * 16, 16)] = zeros

        def p1(i, c):
            u = _sc_key16(row_v[pl.ds(i * 16, 16)])
            b = (u >> 24).astype(jnp.int32)
            plsc.addupdate_scatter(hist_v, [b], ones)
            return c

        lax.fori_loop(0, NV, p1, 0)

        found = jnp.zeros((16,), jnp.bool_)
        bucket = zeros
        cnt_ge = zeros
        hist_b = zeros
        carry = zeros
        for g in range(15, -1, -1):
            v = hist_v[pl.ds(g * 16, 16)]
            rv = lax.rev(v, (0,))
            cs = plsc.cumsum(rv) + carry
            hit = cs >= K
            npop = plsc.all_reduce_population_count(hit)
            ffs = plsc.all_reduce_ffs(hit)
            cg = _splat(jnp.min(jnp.where(hit, cs, _splat(1 << 30))))
            hb = _splat(jnp.max(jnp.where(lane == ffs, rv, zeros)))
            this_found = npop > 0
            upd = jnp.logical_and(this_found, jnp.logical_not(found))
            bucket = jnp.where(upd, _splat(g * 16 + 15) - ffs, bucket)
            cnt_ge = jnp.where(upd, cg, cnt_ge)
            hist_b = jnp.where(upd, hb, hist_b)
            found = jnp.logical_or(found, this_found)
            carry = carry + _splat(jnp.sum(v))

        need = _splat(K) - (cnt_ge - hist_b)

        def p2(i, off):
            u = _sc_key16(row_v[pl.ds(i * 16, 16)])
            m = (u >> 24).astype(jnp.int32) == bucket
            pos = plsc.cumsum(m.astype(jnp.int32)) - 1 + off
            plsc.store_scatter(
                cand_v, [pos], lax.bitcast_convert_type(u, jnp.int32), mask=m)
            return off + plsc.all_reduce_population_count(m)

        off = lax.fori_loop(0, NV, p2, zeros)
        plsc.store_scatter(cand_v, [off + lane], zeros)
        n_cand = jnp.max(off)
        nvc = (n_cand + 15) // 16

        bucket_hi = lax.shift_left(bucket.astype(jnp.uint32), jnp.uint32(24))

        def bstep(t, tl):
            cand_t = tl | lax.shift_left(
                jnp.uint32(1), jnp.uint32(23) - t.astype(jnp.uint32))
            target = bucket_hi | cand_t

            def cl(i, c):
                uu = lax.bitcast_convert_type(
                    cand_v[pl.ds(i * 16, 16)], jnp.uint32)
                return c + plsc.all_reduce_population_count(uu >= target)

            cnt = lax.fori_loop(0, nvc, cl, zeros)
            return jnp.where(cnt >= need, cand_t, tl)

        tl = lax.fori_loop(0, 24, bstep, jnp.zeros((16,), jnp.uint32))
        key = lax.bitcast_convert_type(bucket_hi | tl, jnp.int32)
        plsc.store_scatter(thr_v, [_splat(r)], key, mask=lane == 0)
        return carry_unused

    lax.fori_loop(0, RPW, do_row, 0)
    pltpu.sync_copy(thr_v, thr_hbm.at[pl.ds(base, RPW)])


def _dec_body(h_ref, t_ref, w_ref, b_ref, o_ref):
    k = pl.program_id(0)

    @pl.when(k == 0)
    def _():
        o_ref[...] = jnp.broadcast_to(b_ref[...], o_ref.shape)

    h = h_ref[...]
    thr = jax.lax.bitcast_convert_type(t_ref[...], jnp.uint32)
    hm = jnp.where(_key(h) >= thr, h, 0.0)
    o_ref[...] += jax.lax.dot_general(
        hm, w_ref[...], (((1,), (1,)), ((), ())),
        preferred_element_type=jnp.float32)


def kernel(x, W_enc, b_enc, W_dec, b_dec):
    h = pl.pallas_call(
        _enc_body,
        grid=(NHIDDEN // HBE,),
        in_specs=[
            pl.BlockSpec((B, NIN), lambda j: (0, 0)),
            pl.BlockSpec((HBE, NIN), lambda j: (j, 0)),
            pl.BlockSpec((1, HBE), lambda j: (0, j)),
        ],
        out_specs=pl.BlockSpec((B, HBE), lambda j: (0, j)),
        out_shape=jax.ShapeDtypeStruct((B, NHIDDEN), jnp.float32),
    )(x, W_enc, b_enc.reshape(1, NHIDDEN))

    thr = pl.kernel(
        _sc_thr_body,
        out_type=jax.ShapeDtypeStruct((B,), jnp.int32),
        mesh=plsc.VectorSubcoreMesh(core_axis_name="c", subcore_axis_name="s"),
        scratch_types=[
            pltpu.VMEM((NHIDDEN,), jnp.float32),
            pltpu.VMEM((256,), jnp.int32),
            pltpu.VMEM((NHIDDEN + 16,), jnp.int32),
            pltpu.VMEM((RPW,), jnp.int32),
        ],
        compiler_params=pltpu.CompilerParams(needs_layout_passes=False),
    )(h.reshape(-1))

    out = pl.pallas_call(
        _dec_body,
        grid=(NHIDDEN // KBD,),
        in_specs=[
            pl.BlockSpec((B, KBD), lambda k: (0, k)),
            pl.BlockSpec((B, 1), lambda k: (0, 0)),
            pl.BlockSpec((NOUT, KBD), lambda k: (0, k)),
            pl.BlockSpec((1, NOUT), lambda k: (0, 0)),
        ],
        out_specs=pl.BlockSpec((B, NOUT), lambda k: (0, 0)),
        out_shape=jax.ShapeDtypeStruct((B, NOUT), jnp.float32),
    )(h, thr.reshape(B, 1), W_dec, b_dec.reshape(1, NOUT))
    return out


# speedup vs baseline: 1.5116x; 1.5116x over previous
import functools

import jax
from jax import lax
import jax.numpy as jnp
from jax.experimental import pallas as pl
from jax.experimental.pallas import tpu as pltpu
from jax.experimental.pallas import tpu_sc as plsc

B = 1024
NIN = 2048
NHIDDEN = 8192
NOUT = 2048
K = 64

HBE = 1024
BRM = 256
KBD = 1024


def _enc_body(x_ref, w_ref, b_ref, o_ref):
    acc = jax.lax.dot_general(
        x_ref[...], w_ref[...], (((1,), (1,)), ((), ())),
        preferred_element_type=jnp.float32)
    o_ref[...] = acc + b_ref[...]


def _key(h):
    iv = jax.lax.bitcast_convert_type(h, jnp.uint32)
    return jnp.where((iv >> 31) != 0, ~iv, iv | jnp.uint32(0x80000000))


NC = 2
NS = 16
NW = NC * NS
RPW = B // NW
NV = NHIDDEN // 16


def _splat(v):
    return jnp.full((16,), v, jnp.int32)


def _sc_key16(v):
    iv = lax.bitcast_convert_type(v, jnp.uint32)
    return jnp.where((iv >> 31) != 0, ~iv, iv | jnp.uint32(0x80000000))


def _sc_scan(hist_ref, thresh, lane, zeros):
    found = jnp.zeros((16,), jnp.bool_)
    bucket = zeros
    cnt_ge = zeros
    hist_b = zeros
    carry = zeros
    for g in range(15, -1, -1):
        v = hist_ref[pl.ds(g * 16, 16)]
        rv = lax.rev(v, (0,))
        cs = plsc.cumsum(rv) + carry
        hit = cs >= thresh
        npop = plsc.all_reduce_population_count(hit)
        ffs = plsc.all_reduce_ffs(hit)
        cg = _splat(jnp.min(jnp.where(hit, cs, _splat(1 << 30))))
        hb = _splat(jnp.max(jnp.where(lane == ffs, rv, zeros)))
        this_found = npop > 0
        upd = jnp.logical_and(this_found, jnp.logical_not(found))
        bucket = jnp.where(upd, _splat(g * 16 + 15) - ffs, bucket)
        cnt_ge = jnp.where(upd, cg, cnt_ge)
        hist_b = jnp.where(upd, hb, hist_b)
        found = jnp.logical_or(found, this_found)
        carry = carry + _splat(jnp.sum(v))
    return bucket, thresh - (cnt_ge - hist_b)


def _sc_thr_body(h_hbm, thr_hbm, rowa_v, rowb_v, hist_v, hist2_v, cand_v,
                 cand2_v, thr_v, sema, semb):
    wid = lax.axis_index("s") * NC + lax.axis_index("c")
    base = wid * RPW
    lane = lax.iota(jnp.int32, 16)
    ones = _splat(1)
    zeros = _splat(0)

    def fetch(ri, dst, sem):
        ri = jnp.minimum(ri, RPW - 1)
        pltpu.make_async_copy(
            h_hbm.at[pl.ds((base + ri) * NHIDDEN, NHIDDEN)], dst, sem).start()

    def process(row_v, r):
        for g in range(16):
            hist_v[pl.ds(g * 16, 16)] = zeros

        def p1(i, c):
            for j in range(8):
                u = _sc_key16(row_v[pl.ds((i * 8 + j) * 16, 16)])
                plsc.addupdate_scatter(
                    hist_v, [(u >> 24).astype(jnp.int32)], ones)
            return c

        lax.fori_loop(0, NV // 8, p1, 0)
        bucket, need = _sc_scan(hist_v, _splat(K), lane, zeros)

        def p2(i, off):
            for j in range(8):
                u = _sc_key16(row_v[pl.ds((i * 8 + j) * 16, 16)])
                m = (u >> 24).astype(jnp.int32) == bucket
                pos = plsc.cumsum(m.astype(jnp.int32)) - 1 + off
                plsc.store_scatter(
                    cand_v, [pos], lax.bitcast_convert_type(u, jnp.int32),
                    mask=m)
                off = off + plsc.all_reduce_population_count(m)
            return off

        off = lax.fori_loop(0, NV // 8, p2, zeros)
        for j in range(4):
            plsc.store_scatter(cand_v, [off + lane + _splat(16 * j)], zeros)
        nv4 = (jnp.max(off) + 63) // 64

        for g in range(16):
            hist2_v[pl.ds(g * 16, 16)] = zeros

        def p3a(i, c):
            for j in range(4):
                u = lax.bitcast_convert_type(
                    cand_v[pl.ds((i * 4 + j) * 16, 16)], jnp.uint32)
                b2 = ((u >> 16) & jnp.uint32(0xFF)).astype(jnp.int32)
                m = (u >> 24).astype(jnp.int32) == bucket
                plsc.addupdate_scatter(hist2_v, [b2], ones, mask=m)
            return c

        lax.fori_loop(0, nv4, p3a, 0)
        bucket2, need2 = _sc_scan(hist2_v, need, lane, zeros)

        for j in range(4):
            cand2_v[pl.ds(16 * j, 16)] = zeros

        def p3b(i, off):
            for j in range(4):
                u = lax.bitcast_convert_type(
                    cand_v[pl.ds((i * 4 + j) * 16, 16)], jnp.uint32)
                m = jnp.logical_and(
                    (u >> 24).astype(jnp.int32) == bucket,
                    ((u >> 16) & jnp.uint32(0xFF)).astype(jnp.int32) == bucket2)
                pos = plsc.cumsum(m.astype(jnp.int32)) - 1 + off
                plsc.store_scatter(
                    cand2_v, [pos], lax.bitcast_convert_type(u, jnp.int32),
                    mask=m)
                off = off + plsc.all_reduce_population_count(m)
            return off

        off2 = lax.fori_loop(0, nv4, p3b, zeros)
        plsc.store_scatter(cand2_v, [off2 + lane], zeros)
        n2 = jnp.max(off2)

        prefix = lax.shift_left(bucket.astype(jnp.uint32), jnp.uint32(24)) | \
            lax.shift_left(bucket2.astype(jnp.uint32), jnp.uint32(16))

        def bs_fast():
            cs = [lax.bitcast_convert_type(cand2_v[pl.ds(16 * j, 16)],
                                           jnp.uint32) for j in range(4)]

            def bstep(t, tl):
                cand_t = tl | lax.shift_left(
                    jnp.uint32(1), jnp.uint32(15) - t.astype(jnp.uint32))
                target = prefix | cand_t
                cnt = zeros
                for c in cs:
                    cnt = cnt + plsc.all_reduce_population_count(c >= target)
                return jnp.where(cnt >= need2, cand_t, tl)

            return lax.fori_loop(0, 16, bstep, jnp.zeros((16,), jnp.uint32))

        def bs_slow():
            nvc = (n2 + 15) // 16

            def bstep(t, tl):
                cand_t = tl | lax.shift_left(
                    jnp.uint32(1), jnp.uint32(15) - t.astype(jnp.uint32))
                target = prefix | cand_t

                def cl(i, c):
                    uu = lax.bitcast_convert_type(
                        cand2_v[pl.ds(i * 16, 16)], jnp.uint32)
                    return c + plsc.all_reduce_population_count(uu >= target)

                cnt = lax.fori_loop(0, nvc, cl, zeros)
                return jnp.where(cnt >= need2, cand_t, tl)

            return lax.fori_loop(0, 16, bstep, jnp.zeros((16,), jnp.uint32))

        tl = lax.cond(n2 <= 64, bs_fast, bs_slow)
        key = lax.bitcast_convert_type(prefix | tl, jnp.int32)
        plsc.store_scatter(thr_v, [_splat(r)], key, mask=lane == 0)

    fetch(0, rowa_v, sema)
    def pair(p, c):
        fetch(2 * p + 1, rowb_v, semb)
        pltpu.make_async_copy(h_hbm.at[pl.ds(0, NHIDDEN)], rowa_v, sema).wait()
        process(rowa_v, 2 * p)
        fetch(2 * p + 2, rowa_v, sema)
        pltpu.make_async_copy(h_hbm.at[pl.ds(0, NHIDDEN)], rowb_v, semb).wait()
        process(rowb_v, 2 * p + 1)
        return c

    lax.fori_loop(0, RPW // 2, pair, 0)
    pltpu.make_async_copy(h_hbm.at[pl.ds(0, NHIDDEN)], rowa_v, sema).wait()
    pltpu.sync_copy(thr_v, thr_hbm.at[pl.ds(base, RPW)])


def _dec_body(h_ref, t_ref, w_ref, b_ref, o_ref):
    k = pl.program_id(0)

    @pl.when(k == 0)
    def _():
        o_ref[...] = jnp.broadcast_to(b_ref[...], o_ref.shape)

    h = h_ref[...]
    thr = jax.lax.bitcast_convert_type(t_ref[...], jnp.uint32)
    hm = jnp.where(_key(h) >= thr, h, 0.0)
    o_ref[...] += jax.lax.dot_general(
        hm, w_ref[...], (((1,), (1,)), ((), ())),
        preferred_element_type=jnp.float32)


def kernel(x, W_enc, b_enc, W_dec, b_dec):
    h = pl.pallas_call(
        _enc_body,
        grid=(NHIDDEN // HBE,),
        in_specs=[
            pl.BlockSpec((B, NIN), lambda j: (0, 0)),
            pl.BlockSpec((HBE, NIN), lambda j: (j, 0)),
            pl.BlockSpec((1, HBE), lambda j: (0, j)),
        ],
        out_specs=pl.BlockSpec((B, HBE), lambda j: (0, j)),
        out_shape=jax.ShapeDtypeStruct((B, NHIDDEN), jnp.float32),
    )(x, W_enc, b_enc.reshape(1, NHIDDEN))

    thr = pl.kernel(
        _sc_thr_body,
        out_type=jax.ShapeDtypeStruct((B,), jnp.int32),
        mesh=plsc.VectorSubcoreMesh(core_axis_name="c", subcore_axis_name="s"),
        scratch_types=[
            pltpu.VMEM((NHIDDEN,), jnp.float32),
            pltpu.VMEM((NHIDDEN,), jnp.float32),
            pltpu.VMEM((256,), jnp.int32),
            pltpu.VMEM((256,), jnp.int32),
            pltpu.VMEM((NHIDDEN + 64,), jnp.int32),
            pltpu.VMEM((NHIDDEN + 16,), jnp.int32),
            pltpu.VMEM((RPW,), jnp.int32),
            pltpu.SemaphoreType.DMA,
            pltpu.SemaphoreType.DMA,
        ],
        compiler_params=pltpu.CompilerParams(needs_layout_passes=False),
    )(h.reshape(-1))

    out = pl.pallas_call(
        _dec_body,
        grid=(NHIDDEN // KBD,),
        in_specs=[
            pl.BlockSpec((B, KBD), lambda k: (0, k)),
            pl.BlockSpec((B, 1), lambda k: (0, 0)),
            pl.BlockSpec((NOUT, KBD), lambda k: (0, k)),
            pl.BlockSpec((1, NOUT), lambda k: (0, 0)),
        ],
        out_specs=pl.BlockSpec((B, NOUT), lambda k: (0, 0)),
        out_shape=jax.ShapeDtypeStruct((B, NOUT), jnp.float32),
    )(h, thr.reshape(B, 1), W_dec, b_dec.reshape(1, NOUT))
    return out
